# baseline (speedup 1.0000x reference)
import functools

import jax
import jax.numpy as jnp
from jax import lax
from jax.experimental import pallas as pl
from jax.experimental.pallas import tpu as pltpu
from jax.experimental.pallas import tpu_sc as plsc

NUM_EMB = 100001
KERNEL_DIM = 64
INPUT_DIM = 16
BATCH = 16384

_RBLK = 4096
_NHB = 13
_HALF = _RBLK * _NHB
_PAD_COLS = 128

_NC = 2
_NS = 16
_NW = _NC * _NS
_B_PER_W = BATCH // _NW
_LANES = 16


def _repack_body(lo_t_ref, hi_t_ref, out_ref):
    out_ref[...] = jnp.concatenate([lo_t_ref[...].T, hi_t_ref[...].T], axis=1)


def _repack(table_t):
    return pl.pallas_call(
        _repack_body,
        grid=(_NHB,),
        in_specs=[
            pl.BlockSpec((KERNEL_DIM, _RBLK), lambda i: (0, i)),
            pl.BlockSpec(
                (KERNEL_DIM, _RBLK), lambda i: (0, jnp.minimum(i + _NHB, 24))
            ),
        ],
        out_specs=pl.BlockSpec((_RBLK, _PAD_COLS), lambda i: (i, 0)),
        out_shape=jax.ShapeDtypeStruct((_HALF, _PAD_COLS), jnp.float32),
    )(table_t, table_t)


def _gather_body(idx_hbm, table_hbm, out_hbm, idx_v, rows_v, sem):
    wid = lax.axis_index("s") * _NC + lax.axis_index("c")
    base = wid * _B_PER_W
    pltpu.sync_copy(idx_hbm.at[pl.ds(base, _B_PER_W)], idx_v)
    pltpu.async_copy(table_hbm.at[idx_v], rows_v, sem).wait()
    pltpu.sync_copy(rows_v, out_hbm.at[pl.ds(base, _B_PER_W)])


_sc_gather = functools.partial(
    pl.kernel,
    out_type=jax.ShapeDtypeStruct((BATCH, _PAD_COLS), jnp.float32),
    mesh=plsc.VectorSubcoreMesh(
        core_axis_name="c", subcore_axis_name="s", num_cores=_NC, num_subcores=_NS
    ),
    scratch_types=[
        pltpu.VMEM((_B_PER_W,), jnp.int32),
        pltpu.VMEM((_B_PER_W, _PAD_COLS), jnp.float32),
        pltpu.SemaphoreType.DMA,
    ],
)(_gather_body)


_BLK = 4096


def _tc_body(
    atom_ref, sel_ref, props_t_ref, wp_ref, bp_ref, wc1_ref, wc2_ref, bc_ref, out_ref
):
    sel = sel_ref[...]
    atom = atom_ref[:, :KERNEL_DIM] * (1.0 - sel) + atom_ref[:, KERNEL_DIM:] * sel
    prop_out_t = (
        lax.dot_general(
            wp_ref[...], props_t_ref[...],
            (((1,), (0,)), ((), ())),
            preferred_element_type=jnp.float32,
        )
        + bp_ref[...]
    )
    atom_part_t = lax.dot_general(
        wc1_ref[...], atom,
        (((1,), (1,)), ((), ())),
        preferred_element_type=jnp.float32,
    )
    comp_part_t = lax.dot_general(
        wc2_ref[...], prop_out_t,
        (((1,), (0,)), ((), ())),
        preferred_element_type=jnp.float32,
    )
    out_ref[...] = atom_part_t + comp_part_t + bc_ref[...]


def _tc_fused(atom_out, sel, props_t, W_prop, b_prop, Wc1, Wc2, b_comp):
    grid = BATCH // _BLK
    return pl.pallas_call(
        _tc_body,
        grid=(grid,),
        in_specs=[
            pl.BlockSpec((_BLK, _PAD_COLS), lambda i: (i, 0)),
            pl.BlockSpec((_BLK, 1), lambda i: (i, 0)),
            pl.BlockSpec((INPUT_DIM, _BLK), lambda i: (0, i)),
            pl.BlockSpec((KERNEL_DIM, INPUT_DIM), lambda i: (0, 0)),
            pl.BlockSpec((KERNEL_DIM, 1), lambda i: (0, 0)),
            pl.BlockSpec((KERNEL_DIM, KERNEL_DIM), lambda i: (0, 0)),
            pl.BlockSpec((KERNEL_DIM, KERNEL_DIM), lambda i: (0, 0)),
            pl.BlockSpec((KERNEL_DIM, 1), lambda i: (0, 0)),
        ],
        out_specs=pl.BlockSpec((KERNEL_DIM, _BLK), lambda i: (0, i)),
        out_shape=jax.ShapeDtypeStruct((KERNEL_DIM, BATCH), jnp.float32),
    )(atom_out, sel, props_t, W_prop, b_prop, Wc1, Wc2, b_comp)


def kernel(props, atom_type_ids, atom_table, W_prop, b_prop, W_comp, b_comp):
    ids = atom_type_ids.astype(jnp.int32)
    table_pack = _repack(atom_table.T)
    hi = ids >= _HALF
    pidx = jnp.where(hi, ids - _HALF, ids)
    atom_out = _sc_gather(pidx, table_pack)
    sel = hi.astype(jnp.float32).reshape(BATCH, 1)
    Wc1 = W_comp[:, :KERNEL_DIM]
    Wc2 = W_comp[:, KERNEL_DIM:]
    out_t = _tc_fused(
        atom_out,
        sel,
        props.T,
        W_prop,
        b_prop.reshape(KERNEL_DIM, 1),
        Wc1,
        Wc2,
        b_comp.reshape(KERNEL_DIM, 1),
    )
    return out_t.T


# speedup vs baseline: 1.0941x; 1.0941x over previous
import functools

import jax
import jax.numpy as jnp
from jax import lax
from jax.experimental import pallas as pl
from jax.experimental.pallas import tpu as pltpu
from jax.experimental.pallas import tpu_sc as plsc

NUM_EMB = 100001
KERNEL_DIM = 64
INPUT_DIM = 16
BATCH = 16384

_RBLK = 4096
_NHB = 13
_HALF = _RBLK * _NHB
_PAD_COLS = 128

_NC = 2
_NS = 16
_NW = _NC * _NS
_B_PER_W = BATCH // _NW
_LANES = 16


def _repack_body(lo_t_ref, hi_t_ref, out_ref):
    out_ref[...] = jnp.concatenate([lo_t_ref[...].T, hi_t_ref[...].T], axis=1)


def _repack(table_t):
    return pl.pallas_call(
        _repack_body,
        grid=(_NHB,),
        in_specs=[
            pl.BlockSpec((KERNEL_DIM, _RBLK), lambda i: (0, i)),
            pl.BlockSpec(
                (KERNEL_DIM, _RBLK), lambda i: (0, jnp.minimum(i + _NHB, 24))
            ),
        ],
        out_specs=pl.BlockSpec((_RBLK, _PAD_COLS), lambda i: (i, 0)),
        out_shape=jax.ShapeDtypeStruct((_HALF, _PAD_COLS), jnp.float32),
    )(table_t, table_t)


def _gather_body(idx_hbm, table_hbm, out_hbm, idx_v, rows_v, sem):
    wid = lax.axis_index("s") * _NC + lax.axis_index("c")
    base = wid * _B_PER_W
    pltpu.sync_copy(idx_hbm.at[pl.ds(base, _B_PER_W)], idx_v)
    pltpu.async_copy(table_hbm.at[idx_v], rows_v, sem).wait()
    pltpu.sync_copy(rows_v, out_hbm.at[pl.ds(base, _B_PER_W)])


_sc_gather = functools.partial(
    pl.kernel,
    out_type=jax.ShapeDtypeStruct((BATCH, _PAD_COLS), jnp.float32),
    mesh=plsc.VectorSubcoreMesh(
        core_axis_name="c", subcore_axis_name="s", num_cores=_NC, num_subcores=_NS
    ),
    scratch_types=[
        pltpu.VMEM((_B_PER_W,), jnp.int32),
        pltpu.VMEM((_B_PER_W, _PAD_COLS), jnp.float32),
        pltpu.SemaphoreType.DMA,
    ],
)(_gather_body)


_BLK = 4096


def _tc_body(
    atom_ref, sel_ref, props_t_ref, wp_ref, bp_ref, wc1_ref, wc2_ref, bc_ref, out_ref
):
    sel = sel_ref[...].reshape(_BLK, 1)
    atom = atom_ref[:, :KERNEL_DIM] * (1.0 - sel) + atom_ref[:, KERNEL_DIM:] * sel
    prop_out_t = (
        lax.dot_general(
            wp_ref[...], props_t_ref[...],
            (((1,), (0,)), ((), ())),
            preferred_element_type=jnp.float32,
        )
        + bp_ref[...]
    )
    atom_part_t = lax.dot_general(
        wc1_ref[...], atom,
        (((1,), (1,)), ((), ())),
        preferred_element_type=jnp.float32,
    )
    comp_part_t = lax.dot_general(
        wc2_ref[...], prop_out_t,
        (((1,), (0,)), ((), ())),
        preferred_element_type=jnp.float32,
    )
    out_ref[...] = atom_part_t + comp_part_t + bc_ref[...]


def _tc_fused(atom_out, sel, props_t, W_prop, b_prop, Wc1, Wc2, b_comp):
    grid = BATCH // _BLK
    return pl.pallas_call(
        _tc_body,
        grid=(grid,),
        in_specs=[
            pl.BlockSpec((_BLK, _PAD_COLS), lambda i: (i, 0)),
            pl.BlockSpec((_BLK,), lambda i: (i,)),
            pl.BlockSpec((INPUT_DIM, _BLK), lambda i: (0, i)),
            pl.BlockSpec((KERNEL_DIM, INPUT_DIM), lambda i: (0, 0)),
            pl.BlockSpec((KERNEL_DIM, 1), lambda i: (0, 0)),
            pl.BlockSpec((KERNEL_DIM, KERNEL_DIM), lambda i: (0, 0)),
            pl.BlockSpec((KERNEL_DIM, KERNEL_DIM), lambda i: (0, 0)),
            pl.BlockSpec((KERNEL_DIM, 1), lambda i: (0, 0)),
        ],
        out_specs=pl.BlockSpec((KERNEL_DIM, _BLK), lambda i: (0, i)),
        out_shape=jax.ShapeDtypeStruct((KERNEL_DIM, BATCH), jnp.float32),
    )(atom_out, sel, props_t, W_prop, b_prop, Wc1, Wc2, b_comp)


def kernel(props, atom_type_ids, atom_table, W_prop, b_prop, W_comp, b_comp):
    ids = atom_type_ids.astype(jnp.int32)
    table_pack = _repack(atom_table.T)
    hi = ids >= _HALF
    pidx = jnp.where(hi, ids - _HALF, ids)
    atom_out = _sc_gather(pidx, table_pack)
    sel = hi.astype(jnp.float32)
    Wc1 = W_comp[:, :KERNEL_DIM]
    Wc2 = W_comp[:, KERNEL_DIM:]
    out_t = _tc_fused(
        atom_out,
        sel,
        props.T,
        W_prop,
        b_prop.reshape(KERNEL_DIM, 1),
        Wc1,
        Wc2,
        b_comp.reshape(KERNEL_DIM, 1),
    )
    return out_t.T


# speedup vs baseline: 1.1091x; 1.0137x over previous
import functools

import jax
import jax.numpy as jnp
from jax import lax
from jax.experimental import pallas as pl
from jax.experimental.pallas import tpu as pltpu
from jax.experimental.pallas import tpu_sc as plsc

NUM_EMB = 100001
KERNEL_DIM = 64
INPUT_DIM = 16
BATCH = 16384

_RBLK = 8192
_NHB = 7
_HALF = _RBLK * _NHB
_LAST_IN_BLK = 12
_PAD_COLS = 128

_NC = 2
_NS = 16
_NW = _NC * _NS
_B_PER_W = BATCH // _NW
_LANES = 16


def _repack_body(lo_t_ref, hi_t_ref, out_ref):
    out_ref[...] = jnp.concatenate([lo_t_ref[...].T, hi_t_ref[...].T], axis=1)


def _repack(table_t):
    return pl.pallas_call(
        _repack_body,
        grid=(_NHB,),
        in_specs=[
            pl.BlockSpec((KERNEL_DIM, _RBLK), lambda i: (0, i)),
            pl.BlockSpec(
                (KERNEL_DIM, _RBLK),
                lambda i: (0, jnp.minimum(i + _NHB, _LAST_IN_BLK)),
            ),
        ],
        out_specs=pl.BlockSpec((_RBLK, _PAD_COLS), lambda i: (i, 0)),
        out_shape=jax.ShapeDtypeStruct((_HALF, _PAD_COLS), jnp.float32),
    )(table_t, table_t)


def _gather_body(idx_hbm, table_hbm, out_hbm, idx_v, rows_v, sem):
    wid = lax.axis_index("s") * _NC + lax.axis_index("c")
    base = wid * _B_PER_W
    pltpu.sync_copy(idx_hbm.at[pl.ds(base, _B_PER_W)], idx_v)
    pltpu.async_copy(table_hbm.at[idx_v], rows_v, sem).wait()
    pltpu.sync_copy(rows_v, out_hbm.at[pl.ds(base, _B_PER_W)])


_sc_gather = functools.partial(
    pl.kernel,
    out_type=jax.ShapeDtypeStruct((BATCH, _PAD_COLS), jnp.float32),
    mesh=plsc.VectorSubcoreMesh(
        core_axis_name="c", subcore_axis_name="s", num_cores=_NC, num_subcores=_NS
    ),
    scratch_types=[
        pltpu.VMEM((_B_PER_W,), jnp.int32),
        pltpu.VMEM((_B_PER_W, _PAD_COLS), jnp.float32),
        pltpu.SemaphoreType.DMA,
    ],
)(_gather_body)


_BLK = 8192


def _tc_body(
    atom_ref, sel_ref, props_t_ref, wp_ref, bp_ref, wc1_ref, wc2_ref, bc_ref, out_ref
):
    sel = sel_ref[...].reshape(_BLK, 1)
    atom = atom_ref[:, :KERNEL_DIM] * (1.0 - sel) + atom_ref[:, KERNEL_DIM:] * sel
    prop_out_t = (
        lax.dot_general(
            wp_ref[...], props_t_ref[...],
            (((1,), (0,)), ((), ())),
            preferred_element_type=jnp.float32,
        )
        + bp_ref[...]
    )
    atom_part_t = lax.dot_general(
        wc1_ref[...], atom,
        (((1,), (1,)), ((), ())),
        preferred_element_type=jnp.float32,
    )
    comp_part_t = lax.dot_general(
        wc2_ref[...], prop_out_t,
        (((1,), (0,)), ((), ())),
        preferred_element_type=jnp.float32,
    )
    out_ref[...] = atom_part_t + comp_part_t + bc_ref[...]


def _tc_fused(atom_out, sel, props_t, W_prop, b_prop, Wc1, Wc2, b_comp):
    grid = BATCH // _BLK
    return pl.pallas_call(
        _tc_body,
        grid=(grid,),
        in_specs=[
            pl.BlockSpec((_BLK, _PAD_COLS), lambda i: (i, 0)),
            pl.BlockSpec((_BLK,), lambda i: (i,)),
            pl.BlockSpec((INPUT_DIM, _BLK), lambda i: (0, i)),
            pl.BlockSpec((KERNEL_DIM, INPUT_DIM), lambda i: (0, 0)),
            pl.BlockSpec((KERNEL_DIM, 1), lambda i: (0, 0)),
            pl.BlockSpec((KERNEL_DIM, KERNEL_DIM), lambda i: (0, 0)),
            pl.BlockSpec((KERNEL_DIM, KERNEL_DIM), lambda i: (0, 0)),
            pl.BlockSpec((KERNEL_DIM, 1), lambda i: (0, 0)),
        ],
        out_specs=pl.BlockSpec((KERNEL_DIM, _BLK), lambda i: (0, i)),
        out_shape=jax.ShapeDtypeStruct((KERNEL_DIM, BATCH), jnp.float32),
    )(atom_out, sel, props_t, W_prop, b_prop, Wc1, Wc2, b_comp)


def kernel(props, atom_type_ids, atom_table, W_prop, b_prop, W_comp, b_comp):
    ids = atom_type_ids.astype(jnp.int32)
    table_pack = _repack(atom_table.T)
    hi = ids >= _HALF
    pidx = jnp.where(hi, ids - _HALF, ids)
    atom_out = _sc_gather(pidx, table_pack)
    sel = hi.astype(jnp.float32)
    Wc1 = W_comp[:, :KERNEL_DIM]
    Wc2 = W_comp[:, KERNEL_DIM:]
    out_t = _tc_fused(
        atom_out,
        sel,
        props.T,
        W_prop,
        b_prop.reshape(KERNEL_DIM, 1),
        Wc1,
        Wc2,
        b_comp.reshape(KERNEL_DIM, 1),
    )
    return out_t.T


# speedup vs baseline: 1.1330x; 1.0215x over previous
import functools

import jax
import jax.numpy as jnp
from jax import lax
from jax.experimental import pallas as pl
from jax.experimental.pallas import tpu as pltpu
from jax.experimental.pallas import tpu_sc as plsc

NUM_EMB = 100001
KERNEL_DIM = 64
INPUT_DIM = 16
BATCH = 16384

_RBLK = 8192
_NHB = 7
_HALF = _RBLK * _NHB
_LAST_IN_BLK = 12
_PAD_COLS = 128

_NC = 2
_NS = 16
_NW = _NC * _NS
_B_PER_W = BATCH // _NW
_LANES = 16


def _repack_body(lo_t_ref, hi_t_ref, out_ref):
    out_ref[...] = jnp.concatenate([lo_t_ref[...].T, hi_t_ref[...].T], axis=1)


def _repack(table_t):
    return pl.pallas_call(
        _repack_body,
        grid=(_NHB,),
        in_specs=[
            pl.BlockSpec((KERNEL_DIM, _RBLK), lambda i: (0, i)),
            pl.BlockSpec(
                (KERNEL_DIM, _RBLK),
                lambda i: (0, jnp.minimum(i + _NHB, _LAST_IN_BLK)),
            ),
        ],
        out_specs=pl.BlockSpec((_RBLK, _PAD_COLS), lambda i: (i, 0)),
        out_shape=jax.ShapeDtypeStruct((_HALF, _PAD_COLS), jnp.float32),
    )(table_t, table_t)


def _gather_body(idx_hbm, table_hbm, out_hbm, idx_v, pidx_v, rows_v, sem):
    wid = lax.axis_index("s") * _NC + lax.axis_index("c")
    base = wid * _B_PER_W
    pltpu.sync_copy(idx_hbm.at[pl.ds(base, _B_PER_W)], idx_v)

    def remap(j, carry):
        v = idx_v[pl.ds(j * _LANES, _LANES)]
        pidx_v[pl.ds(j * _LANES, _LANES)] = jnp.where(v >= _HALF, v - _HALF, v)
        return carry

    lax.fori_loop(0, _B_PER_W // _LANES, remap, 0)
    pltpu.async_copy(table_hbm.at[pidx_v], rows_v, sem).wait()
    pltpu.sync_copy(rows_v, out_hbm.at[pl.ds(base, _B_PER_W)])


_sc_gather = functools.partial(
    pl.kernel,
    out_type=jax.ShapeDtypeStruct((BATCH, _PAD_COLS), jnp.float32),
    mesh=plsc.VectorSubcoreMesh(
        core_axis_name="c", subcore_axis_name="s", num_cores=_NC, num_subcores=_NS
    ),
    scratch_types=[
        pltpu.VMEM((_B_PER_W,), jnp.int32),
        pltpu.VMEM((_B_PER_W,), jnp.int32),
        pltpu.VMEM((_B_PER_W, _PAD_COLS), jnp.float32),
        pltpu.SemaphoreType.DMA,
    ],
)(_gather_body)


_BLK = 8192


def _tc_body(
    atom_ref, sel_ref, props_t_ref, wp_ref, bp_ref, wc1_ref, wc2_ref, bc_ref, out_ref
):
    sel = (sel_ref[...] >= _HALF).astype(jnp.float32).reshape(_BLK, 1)
    atom = atom_ref[:, :KERNEL_DIM] * (1.0 - sel) + atom_ref[:, KERNEL_DIM:] * sel
    prop_out_t = (
        lax.dot_general(
            wp_ref[...], props_t_ref[...],
            (((1,), (0,)), ((), ())),
            preferred_element_type=jnp.float32,
        )
        + bp_ref[...]
    )
    atom_part_t = lax.dot_general(
        wc1_ref[...], atom,
        (((1,), (1,)), ((), ())),
        preferred_element_type=jnp.float32,
    )
    comp_part_t = lax.dot_general(
        wc2_ref[...], prop_out_t,
        (((1,), (0,)), ((), ())),
        preferred_element_type=jnp.float32,
    )
    out_ref[...] = atom_part_t + comp_part_t + bc_ref[...]


def _tc_fused(atom_out, sel, props_t, W_prop, b_prop, Wc1, Wc2, b_comp):
    grid = BATCH // _BLK
    return pl.pallas_call(
        _tc_body,
        grid=(grid,),
        in_specs=[
            pl.BlockSpec((_BLK, _PAD_COLS), lambda i: (i, 0)),
            pl.BlockSpec((_BLK,), lambda i: (i,)),
            pl.BlockSpec((INPUT_DIM, _BLK), lambda i: (0, i)),
            pl.BlockSpec((KERNEL_DIM, INPUT_DIM), lambda i: (0, 0)),
            pl.BlockSpec((KERNEL_DIM, 1), lambda i: (0, 0)),
            pl.BlockSpec((KERNEL_DIM, KERNEL_DIM), lambda i: (0, 0)),
            pl.BlockSpec((KERNEL_DIM, KERNEL_DIM), lambda i: (0, 0)),
            pl.BlockSpec((KERNEL_DIM, 1), lambda i: (0, 0)),
        ],
        out_specs=pl.BlockSpec((KERNEL_DIM, _BLK), lambda i: (0, i)),
        out_shape=jax.ShapeDtypeStruct((KERNEL_DIM, BATCH), jnp.float32),
    )(atom_out, sel, props_t, W_prop, b_prop, Wc1, Wc2, b_comp)


def kernel(props, atom_type_ids, atom_table, W_prop, b_prop, W_comp, b_comp):
    ids = atom_type_ids.astype(jnp.int32)
    table_pack = _repack(atom_table.T)
    atom_out = _sc_gather(ids, table_pack)
    Wc1 = W_comp[:, :KERNEL_DIM]
    Wc2 = W_comp[:, KERNEL_DIM:]
    out_t = _tc_fused(
        atom_out,
        ids,
        props.T,
        W_prop,
        b_prop.reshape(KERNEL_DIM, 1),
        Wc1,
        Wc2,
        b_comp.reshape(KERNEL_DIM, 1),
    )
    return out_t.T
